# baseline (speedup 1.0000x reference)
import dataclasses
import functools

import jax
import jax.numpy as jnp
from jax import lax
from jax.experimental import pallas as pl
from jax.experimental.pallas import tpu as pltpu
from jax.experimental.pallas import tpu_sc as plsc


def kernel(x, edge_index, edge_weight, traj_seqs, seq_lengths, W, b):
    N, F = x.shape
    E = edge_index.shape[1]
    B, L = traj_seqs.shape
    NC, NS = 2, 16
    NW = NC * NS
    CHUNK = 128
    NT = 10240
    STRIPE = NT // NS
    NPAD = 64
    FCH = F // 16

    per_w = -(-E // (NW * CHUNK)) * CHUNK
    NCH = per_w // CHUNK
    pad = per_w * NW - E
    HF = F // 2
    per_s = -(-E // (NS * CHUNK * 2)) * CHUNK * 2
    NCH2 = per_s // CHUNK
    pad2 = per_s * NS - E

    TOK = B * L
    TPW = TOK // NW
    TCH = TPW // CHUNK

    i32 = jnp.int32
    f32 = jnp.float32

    src = edge_index[0].astype(i32)
    dst = edge_index[1].astype(i32)
    w = edge_weight.astype(f32)
    pad_idx = jnp.arange(pad, dtype=i32) % N
    src3 = jnp.concatenate([src, pad_idx]).reshape(NW, NCH, CHUNK)
    dst3 = jnp.concatenate([dst, pad_idx]).reshape(NW, NCH, CHUNK)
    w_p = jnp.concatenate([w, jnp.zeros((pad,), f32)])
    w3 = w_p.reshape(NW, NCH, CHUNK)
    pad_idx2 = jnp.arange(pad2, dtype=i32) % N
    src2 = jnp.concatenate([src, pad_idx2]).reshape(NS, NCH2, CHUNK)
    src2b = jnp.stack([src2, src2 + NT])
    dst2 = jnp.concatenate([dst, pad_idx2]).reshape(NS, NCH2, CHUNK)
    w2 = jnp.concatenate([w, jnp.zeros((pad2,), f32)]).reshape(NS, NCH2, CHUNK)

    zdeg = jnp.zeros((STRIPE, 16), f32)
    zrow = jnp.zeros((STRIPE, HF), f32)
    x_p = jnp.concatenate([x, jnp.zeros((NT - N, F), f32)])

    mesh = plsc.VectorSubcoreMesh(core_axis_name="c", subcore_axis_name="s")
    _cp = pltpu.CompilerParams()
    if "needs_layout_passes" in pltpu.CompilerParams.__dataclass_fields__:
        _cp = dataclasses.replace(_cp, needs_layout_passes=False)
    _cp_sc = dataclasses.replace(_cp, use_tc_tiling_on_sc=False)

    @functools.partial(
        pl.kernel,
        out_type=jax.ShapeDtypeStruct((NC, NT, 16), f32),
        mesh=mesh,
        compiler_params=_cp_sc,
        scratch_types=[
            pltpu.VMEM((NCH, CHUNK), i32),
            pltpu.VMEM((NCH, CHUNK), f32),
            pltpu.VMEM((CHUNK, 16), f32),
            pltpu.VMEM_SHARED((NT, 16), f32),
        ],
    )
    def _deg(dst_hbm, w_hbm, z_hbm, out_hbm, dst_v, w_v, buf, deg_sh):
        c = lax.axis_index("c")
        s = lax.axis_index("s")
        wid = c * NS + s
        pltpu.sync_copy(dst_hbm.at[wid], dst_v)
        pltpu.sync_copy(w_hbm.at[wid], w_v)
        pltpu.sync_copy(z_hbm.at[pl.ds(0, CHUNK)], buf)
        pltpu.sync_copy(z_hbm, deg_sh.at[pl.ds(s * STRIPE, STRIPE)])
        plsc.subcore_barrier()

        zero16 = jnp.zeros((16,), i32)
        iota16 = lax.broadcasted_iota(i32, (16,), 0)

        @pl.loop(0, NCH)
        def _(j):
            for g16 in range(CHUNK // 16):
                wv = w_v[j, pl.ds(g16 * 16, 16)]
                plsc.store_scatter(buf, [iota16 + g16 * 16, zero16], wv)
            pltpu.sync_copy(buf, deg_sh.at[dst_v.at[j]], add=True)

        plsc.subcore_barrier()
        pltpu.sync_copy(deg_sh.at[pl.ds(s * STRIPE, STRIPE)],
                        out_hbm.at[c].at[pl.ds(s * STRIPE, STRIPE)])

    def _mm_body(x_ref, w_ref, h_ref):
        h_ref[...] = jnp.dot(x_ref[...], w_ref[...],
                             preferred_element_type=f32)

    _mm = pl.pallas_call(
        _mm_body, out_shape=jax.ShapeDtypeStruct((NT, F), f32))

    def _disg_body(dp_ref, h_ref, dis_ref, g_ref):
        deg = dp_ref[0, :, 0] + dp_ref[1, :, 0] + 1.0
        dis = lax.rsqrt(deg)
        dis_ref[...] = dis.reshape(NT, 1)
        hh = h_ref[...] * dis[:, None]
        g_ref[pl.ds(0, NT), :] = hh[:, 0:HF]
        g_ref[pl.ds(NT, NT), :] = hh[:, HF:F]

    _disg = pl.pallas_call(
        _disg_body,
        out_shape=(jax.ShapeDtypeStruct((NT, 1), f32),
                   jax.ShapeDtypeStruct((2 * NT, HF), f32)))

    @functools.partial(
        pl.kernel,
        out_type=jax.ShapeDtypeStruct((NC, NT, HF), f32),
        mesh=mesh,
        compiler_params=_cp_sc,
        scratch_types=[
            pltpu.VMEM((NCH2, CHUNK), i32),
            pltpu.VMEM((NCH2, CHUNK), i32),
            pltpu.VMEM((NCH2, CHUNK), f32),
            pltpu.VMEM((CHUNK, HF), f32),
            pltpu.VMEM((CHUNK, HF), f32),
            pltpu.SMEM((CHUNK,), f32),
            pltpu.SMEM((CHUNK,), f32),
            pltpu.VMEM_SHARED((NT, HF), f32),
            pltpu.SemaphoreType.DMA,
            pltpu.SemaphoreType.DMA,
        ],
    )
    def _scat(srcb_hbm, dst_hbm, w_hbm, g_hbm, z_hbm, out_hbm,
              src_v, dst_v, w_v, buf0, buf1, wsm0, wsm1, acc_sh, sem0, sem1):
        c = lax.axis_index("c")
        s = lax.axis_index("s")
        pltpu.sync_copy(srcb_hbm.at[c].at[s], src_v)
        pltpu.sync_copy(dst_hbm.at[s], dst_v)
        pltpu.sync_copy(w_hbm.at[s], w_v)
        pltpu.sync_copy(z_hbm, acc_sh.at[pl.ds(s * STRIPE, STRIPE)])
        plsc.subcore_barrier()

        def scale(buf, wsm, j):
            @pl.loop(0, CHUNK // 16)
            def _(g):
                w16 = w_v[j, pl.ds(g * 16, 16)]
                for rr in range(16):
                    wv = jnp.full((16,), w16[rr], dtype=f32)
                    row = g * 16 + rr
                    for g16 in range(HF // 16):
                        sl = pl.ds(g16 * 16, 16)
                        buf[row, sl] = buf[row, sl] * wv

        pltpu.async_copy(g_hbm.at[src_v.at[0]], buf0, sem0)

        @pl.loop(0, NCH2 // 2)
        def _(k):
            j = k * 2
            pltpu.async_copy(g_hbm.at[src_v.at[j + 1]], buf1, sem1)
            pltpu.make_async_copy(g_hbm.at[src_v.at[j]], buf0, sem0).wait()
            scale(buf0, wsm0, j)
            pltpu.sync_copy(buf0, acc_sh.at[dst_v.at[j]], add=True)

            @pl.when(j + 2 < NCH2)
            def _():
                pltpu.async_copy(g_hbm.at[src_v.at[j + 2]], buf0, sem0)

            pltpu.make_async_copy(g_hbm.at[src_v.at[j + 1]], buf1, sem1).wait()
            scale(buf1, wsm1, j + 1)
            pltpu.sync_copy(buf1, acc_sh.at[dst_v.at[j + 1]], add=True)

        plsc.subcore_barrier()
        pltpu.sync_copy(acc_sh.at[pl.ds(s * STRIPE, STRIPE)],
                        out_hbm.at[c].at[pl.ds(s * STRIPE, STRIPE)])

    def _emb_body(ap_ref, h_ref, dis_ref, b_ref, tr_ref, len_ref,
                  t_ref, o_ref):
        acc = jnp.concatenate([ap_ref[0], ap_ref[1]], axis=1)
        dis = dis_ref[...]
        emb = acc * dis + h_ref[...] * (dis * dis) + b_ref[...]
        t_ref[...] = jnp.maximum(emb, 0.0)
        t_ref[pl.ds(N, NPAD), :] = jnp.zeros((NPAD, F), f32)
        pos = lax.broadcasted_iota(i32, (B, L), 1)
        row = lax.broadcasted_iota(i32, (B, L), 0)
        flat = row * L + pos
        o_ref[...] = jnp.where(pos < len_ref[...], tr_ref[...],
                               N + (flat & (NPAD - 1)))

    _emb = pl.pallas_call(
        _emb_body, out_shape=(jax.ShapeDtypeStruct((NT, F), f32),
                              jax.ShapeDtypeStruct((B, L), i32)))

    @functools.partial(
        pl.kernel,
        out_type=jax.ShapeDtypeStruct((TOK, F), f32),
        mesh=mesh,
        scratch_types=[
            pltpu.VMEM((TCH, CHUNK), i32),
            pltpu.VMEM((CHUNK, F), f32),
            pltpu.VMEM((CHUNK, F), f32),
            pltpu.SemaphoreType.DMA,
            pltpu.SemaphoreType.DMA,
        ],
    )
    def _gat(t_hbm, idx_hbm, out_hbm, idx_v, buf0, buf1, sem0, sem1):
        c = lax.axis_index("c")
        s = lax.axis_index("s")
        wid = c * NS + s
        pltpu.sync_copy(idx_hbm.at[wid], idx_v)
        base = wid * TPW
        pltpu.async_copy(t_hbm.at[idx_v.at[0]], buf0, sem0)

        @pl.loop(0, TCH // 2)
        def _(k):
            j = k * 2
            pltpu.async_copy(t_hbm.at[idx_v.at[j + 1]], buf1, sem1)
            pltpu.make_async_copy(t_hbm.at[idx_v.at[j]], buf0, sem0).wait()
            pltpu.sync_copy(buf0, out_hbm.at[pl.ds(base + j * CHUNK, CHUNK)])

            @pl.when(j + 2 < TCH)
            def _():
                pltpu.async_copy(t_hbm.at[idx_v.at[j + 2]], buf0, sem0)

            pltpu.make_async_copy(t_hbm.at[idx_v.at[j + 1]], buf1, sem1).wait()
            pltpu.sync_copy(buf1, out_hbm.at[pl.ds(base + (j + 1) * CHUNK, CHUNK)])

    deg_parts = _deg(dst3, w3, zdeg)
    h = _mm(x_p, W)
    dis, g = _disg(deg_parts, h)
    acc_parts = _scat(src2b, dst2, w2, g, zrow)
    table, midx = _emb(acc_parts, h, dis, b.reshape(1, F),
                       traj_seqs.astype(i32),
                       seq_lengths.astype(i32).reshape(B, 1))
    out = _gat(table, midx.reshape(NW, TCH, CHUNK))
    return out.reshape(B, L, F), seq_lengths


# speedup vs baseline: 1.8322x; 1.8322x over previous
import dataclasses
import functools

import jax
import jax.numpy as jnp
from jax import lax
from jax.experimental import pallas as pl
from jax.experimental.pallas import tpu as pltpu
from jax.experimental.pallas import tpu_sc as plsc


def kernel(x, edge_index, edge_weight, traj_seqs, seq_lengths, W, b):
    N, F = x.shape
    E = edge_index.shape[1]
    B, L = traj_seqs.shape
    NC, NS = 2, 16
    NW = NC * NS
    CHUNK = 128
    NT = 10240
    STRIPE = NT // NS
    NPAD = 64
    FCH = F // 16

    per_w = -(-E // (NW * CHUNK)) * CHUNK
    NCH = per_w // CHUNK
    pad = per_w * NW - E
    HF = F // 2
    per_s = -(-E // (NS * CHUNK * 2)) * CHUNK * 2
    NCH2 = per_s // CHUNK
    pad2 = per_s * NS - E

    TOK = B * L
    TPW = TOK // NW
    TCH = TPW // CHUNK

    i32 = jnp.int32
    f32 = jnp.float32

    src = edge_index[0].astype(i32)
    dst = edge_index[1].astype(i32)
    w = edge_weight.astype(f32)
    pad_idx = jnp.arange(pad, dtype=i32) % N
    src3 = jnp.concatenate([src, pad_idx]).reshape(NW, NCH, CHUNK)
    dst3 = jnp.concatenate([dst, pad_idx]).reshape(NW, NCH, CHUNK)
    w_p = jnp.concatenate([w, jnp.zeros((pad,), f32)])
    w3 = w_p.reshape(NW, NCH, CHUNK)
    pad_idx2 = jnp.arange(pad2, dtype=i32) % N
    src2 = jnp.concatenate([src, pad_idx2]).reshape(NS, NCH2, CHUNK)
    src2b = jnp.stack([src2, src2 + NT])
    dst2 = jnp.concatenate([dst, pad_idx2]).reshape(NS, NCH2, CHUNK)
    w2 = jnp.concatenate([w, jnp.zeros((pad2,), f32)]).reshape(NS, NCH2, CHUNK)

    zdeg = jnp.zeros((STRIPE, 16), f32)
    zrow = jnp.zeros((STRIPE, HF), f32)
    x_p = jnp.concatenate([x, jnp.zeros((NT - N, F), f32)])

    mesh = plsc.VectorSubcoreMesh(core_axis_name="c", subcore_axis_name="s")
    _cp = pltpu.CompilerParams()
    if "needs_layout_passes" in pltpu.CompilerParams.__dataclass_fields__:
        _cp = dataclasses.replace(_cp, needs_layout_passes=False)
    _cp_sc = dataclasses.replace(_cp, use_tc_tiling_on_sc=False)

    @functools.partial(
        pl.kernel,
        out_type=jax.ShapeDtypeStruct((NC, NT, 16), f32),
        mesh=mesh,
        compiler_params=_cp_sc,
        scratch_types=[
            pltpu.VMEM((NCH, CHUNK), i32),
            pltpu.VMEM((NCH, CHUNK), f32),
            pltpu.VMEM((CHUNK, 16), f32),
            pltpu.VMEM_SHARED((NT, 16), f32),
        ],
    )
    def _deg(dst_hbm, w_hbm, z_hbm, out_hbm, dst_v, w_v, buf, deg_sh):
        c = lax.axis_index("c")
        s = lax.axis_index("s")
        wid = c * NS + s
        pltpu.sync_copy(dst_hbm.at[wid], dst_v)
        pltpu.sync_copy(w_hbm.at[wid], w_v)
        pltpu.sync_copy(z_hbm.at[pl.ds(0, CHUNK)], buf)
        pltpu.sync_copy(z_hbm, deg_sh.at[pl.ds(s * STRIPE, STRIPE)])
        plsc.subcore_barrier()

        zero16 = jnp.zeros((16,), i32)
        iota16 = lax.broadcasted_iota(i32, (16,), 0)

        @pl.loop(0, NCH)
        def _(j):
            for g16 in range(CHUNK // 16):
                wv = w_v[j, pl.ds(g16 * 16, 16)]
                plsc.store_scatter(buf, [iota16 + g16 * 16, zero16], wv)
            pltpu.sync_copy(buf, deg_sh.at[dst_v.at[j]], add=True)

        plsc.subcore_barrier()
        pltpu.sync_copy(deg_sh.at[pl.ds(s * STRIPE, STRIPE)],
                        out_hbm.at[c].at[pl.ds(s * STRIPE, STRIPE)])

    def _mm_body(x_ref, w_ref, h_ref):
        h_ref[...] = jnp.dot(x_ref[...], w_ref[...],
                             preferred_element_type=f32)

    _mm = pl.pallas_call(
        _mm_body, out_shape=jax.ShapeDtypeStruct((NT, F), f32))

    def _disg_body(dp_ref, h_ref, dis_ref, g_ref):
        deg = dp_ref[0, :, 0] + dp_ref[1, :, 0] + 1.0
        dis = lax.rsqrt(deg)
        dis_ref[...] = dis.reshape(NT, 1)
        hh = h_ref[...] * dis[:, None]
        g_ref[pl.ds(0, NT), :] = hh[:, 0:HF]
        g_ref[pl.ds(NT, NT), :] = hh[:, HF:F]

    _disg = pl.pallas_call(
        _disg_body,
        out_shape=(jax.ShapeDtypeStruct((NT, 1), f32),
                   jax.ShapeDtypeStruct((2 * NT, HF), f32)))

    @functools.partial(
        pl.kernel,
        out_type=jax.ShapeDtypeStruct((NC, NT, HF), f32),
        mesh=mesh,
        compiler_params=_cp_sc,
        scratch_types=[
            pltpu.VMEM((NCH2, CHUNK), i32),
            pltpu.VMEM((NCH2, CHUNK), i32),
            pltpu.VMEM((NCH2, CHUNK), f32),
            pltpu.VMEM((CHUNK, HF), f32),
            pltpu.VMEM((CHUNK, HF), f32),
            pltpu.VMEM_SHARED((NT, HF), f32),
            pltpu.SemaphoreType.DMA,
            pltpu.SemaphoreType.DMA,
        ],
    )
    def _scat(srcb_hbm, dst_hbm, w_hbm, g_hbm, z_hbm, out_hbm,
              src_v, dst_v, w_v, buf0, buf1, acc_sh, sem0, sem1):
        c = lax.axis_index("c")
        s = lax.axis_index("s")
        pltpu.sync_copy(srcb_hbm.at[c].at[s], src_v)
        pltpu.sync_copy(dst_hbm.at[s], dst_v)
        pltpu.sync_copy(w_hbm.at[s], w_v)
        pltpu.sync_copy(z_hbm, acc_sh.at[pl.ds(s * STRIPE, STRIPE)])
        plsc.subcore_barrier()

        def scale(buf, j):
            jv = jnp.full((16,), j, dtype=i32)

            @plsc.parallel_loop(0, CHUNK, 1, unroll=4)
            def _(r):
                wv = plsc.load_gather(w_v, [jv, jnp.full((16,), r, dtype=i32)])
                for g16 in range(HF // 16):
                    sl = pl.ds(g16 * 16, 16)
                    buf[r, sl] = buf[r, sl] * wv

        pltpu.async_copy(g_hbm.at[src_v.at[0]], buf0, sem0)

        @pl.loop(0, NCH2 // 2)
        def _(k):
            j = k * 2
            pltpu.async_copy(g_hbm.at[src_v.at[j + 1]], buf1, sem1)
            pltpu.make_async_copy(g_hbm.at[src_v.at[j]], buf0, sem0).wait()
            scale(buf0, j)
            pltpu.sync_copy(buf0, acc_sh.at[dst_v.at[j]], add=True)

            @pl.when(j + 2 < NCH2)
            def _():
                pltpu.async_copy(g_hbm.at[src_v.at[j + 2]], buf0, sem0)

            pltpu.make_async_copy(g_hbm.at[src_v.at[j + 1]], buf1, sem1).wait()
            scale(buf1, j + 1)
            pltpu.sync_copy(buf1, acc_sh.at[dst_v.at[j + 1]], add=True)

        plsc.subcore_barrier()
        pltpu.sync_copy(acc_sh.at[pl.ds(s * STRIPE, STRIPE)],
                        out_hbm.at[c].at[pl.ds(s * STRIPE, STRIPE)])

    def _emb_body(ap_ref, h_ref, dis_ref, b_ref, tr_ref, len_ref,
                  t_ref, o_ref):
        acc = jnp.concatenate([ap_ref[0], ap_ref[1]], axis=1)
        dis = dis_ref[...]
        emb = acc * dis + h_ref[...] * (dis * dis) + b_ref[...]
        t_ref[...] = jnp.maximum(emb, 0.0)
        t_ref[pl.ds(N, NPAD), :] = jnp.zeros((NPAD, F), f32)
        pos = lax.broadcasted_iota(i32, (B, L), 1)
        row = lax.broadcasted_iota(i32, (B, L), 0)
        flat = row * L + pos
        o_ref[...] = jnp.where(pos < len_ref[...], tr_ref[...],
                               N + (flat & (NPAD - 1)))

    _emb = pl.pallas_call(
        _emb_body, out_shape=(jax.ShapeDtypeStruct((NT, F), f32),
                              jax.ShapeDtypeStruct((B, L), i32)))

    @functools.partial(
        pl.kernel,
        out_type=jax.ShapeDtypeStruct((TOK, F), f32),
        mesh=mesh,
        scratch_types=[
            pltpu.VMEM((TCH, CHUNK), i32),
            pltpu.VMEM((CHUNK, F), f32),
            pltpu.VMEM((CHUNK, F), f32),
            pltpu.SemaphoreType.DMA,
            pltpu.SemaphoreType.DMA,
        ],
    )
    def _gat(t_hbm, idx_hbm, out_hbm, idx_v, buf0, buf1, sem0, sem1):
        c = lax.axis_index("c")
        s = lax.axis_index("s")
        wid = c * NS + s
        pltpu.sync_copy(idx_hbm.at[wid], idx_v)
        base = wid * TPW
        pltpu.async_copy(t_hbm.at[idx_v.at[0]], buf0, sem0)

        @pl.loop(0, TCH // 2)
        def _(k):
            j = k * 2
            pltpu.async_copy(t_hbm.at[idx_v.at[j + 1]], buf1, sem1)
            pltpu.make_async_copy(t_hbm.at[idx_v.at[j]], buf0, sem0).wait()
            pltpu.sync_copy(buf0, out_hbm.at[pl.ds(base + j * CHUNK, CHUNK)])

            @pl.when(j + 2 < TCH)
            def _():
                pltpu.async_copy(t_hbm.at[idx_v.at[j + 2]], buf0, sem0)

            pltpu.make_async_copy(t_hbm.at[idx_v.at[j + 1]], buf1, sem1).wait()
            pltpu.sync_copy(buf1, out_hbm.at[pl.ds(base + (j + 1) * CHUNK, CHUNK)])

    deg_parts = _deg(dst3, w3, zdeg)
    h = _mm(x_p, W)
    dis, g = _disg(deg_parts, h)
    acc_parts = _scat(src2b, dst2, w2, g, zrow)
    table, midx = _emb(acc_parts, h, dis, b.reshape(1, F),
                       traj_seqs.astype(i32),
                       seq_lengths.astype(i32).reshape(B, 1))
    out = _gat(table, midx.reshape(NW, TCH, CHUNK))
    return out.reshape(B, L, F), seq_lengths
